# baseline (speedup 1.0000x reference)
import functools

import jax
import jax.numpy as jnp
import numpy as np
from jax import lax
from jax.experimental import pallas as pl
from jax.experimental.pallas import tpu as pltpu
from jax.experimental.pallas import tpu_sc as plsc

N_PTS = 131072
N_LEVELS = 16
FEAT = 2
TBL = 2 ** 19
BASE_RES = 16
PLS = 1.4472692012786865
MASK = TBL - 1
P1 = np.uint32(2654435761).astype(np.int32)
P2 = np.uint32(805459861).astype(np.int32)

NW = 32
PPW = N_PTS // NW
CHUNK = 128
NCHUNK = PPW // CHUNK

_RES = [int(np.floor(BASE_RES * (PLS ** l))) for l in range(N_LEVELS)]


@functools.cache
def _sc_encode_fn():
    mesh = plsc.VectorSubcoreMesh(core_axis_name="c", subcore_axis_name="s")

    @functools.partial(
        pl.kernel,
        mesh=mesh,
        compiler_params=pltpu.CompilerParams(
            needs_layout_passes=False, use_tc_tiling_on_sc=False),
        out_type=jax.ShapeDtypeStruct((N_PTS, 2 * N_LEVELS), jnp.float32),
        scratch_types=[
            pltpu.VMEM((3, PPW), jnp.float32),
            pltpu.VMEM((8, CHUNK), jnp.int32),
            pltpu.VMEM((8 * CHUNK,), jnp.int32),
            pltpu.VMEM((8 * CHUNK,), jnp.float32),
            pltpu.VMEM((8 * CHUNK, 16), jnp.float32),
            pltpu.VMEM((CHUNK, 2 * N_LEVELS), jnp.float32),
            pltpu.SemaphoreType.DMA,
        ],
    )
    def encode(pos_hbm, tab_hbm, out_hbm, pos_v, idx_v, sub_v, w_v, rows_v,
               enc_v, sem):
        cid = lax.axis_index("c")
        sid = lax.axis_index("s")
        wid = sid * 2 + cid
        base = wid * PPW

        pltpu.sync_copy(pos_hbm.at[:, pl.ds(base, PPW)], pos_v)

        lanes = lax.iota(jnp.int32, 16)
        half = lanes >> 1
        fbit = lanes & 1

        def norm_body(g, carry):
            for r in range(3):
                v = pos_v[r, pl.ds(g * 16, 16)]
                pos_v[r, pl.ds(g * 16, 16)] = (v + 1.0) * 0.5
            return carry

        lax.fori_loop(0, PPW // 16, norm_body, 0)

        def chunk_body(ch, carry):
            pbase = ch * CHUNK

            for l in range(N_LEVELS):
                res = float(_RES[l])
                lbase = l * TBL

                def hash_body(g, c2, res=res, lbase=lbase):
                    o = pbase + g * 16
                    ws = []
                    hs = []
                    for r, (mulc, addc) in enumerate(((1, 1), (int(P1), int(P1)), (int(P2), int(P2)))):
                        x = pos_v[r, pl.ds(o, 16)]
                        pos = x * res
                        ipos = pos.astype(jnp.int32)
                        w = pos - ipos.astype(jnp.float32)
                        sw = w * w * (3.0 - 2.0 * w)
                        ws.append((1.0 - sw, sw))
                        h0 = ipos * mulc if r > 0 else ipos
                        hs.append((h0, h0 + addc))
                    for c in range(8):
                        bx, by, bz = (c >> 0) & 1, (c >> 1) & 1, (c >> 2) & 1
                        h = ((hs[0][bx] ^ hs[1][by] ^ hs[2][bz]) & MASK) + lbase
                        idx_v[c, pl.ds(g * 16, 16)] = h >> 3
                        sub_v[pl.ds(c * CHUNK + g * 16, 16)] = h & 7
                        w_v[pl.ds(c * CHUNK + g * 16, 16)] = ws[0][bx] * ws[1][by] * ws[2][bz]
                    return c2

                lax.fori_loop(0, CHUNK // 16, hash_body, 0)

                handles = [
                    pltpu.async_copy(tab_hbm.at[idx_v.at[c]],
                                     rows_v.at[pl.ds(c * CHUNK, CHUNK)], sem)
                    for c in range(8)
                ]
                for h in handles:
                    h.wait()

                def acc_body(pg, c2, l=l):
                    pidx = pg * 8 + half
                    cidx = fbit + 2 * l
                    for c in range(8):
                        ridx = pidx + (c * CHUNK)
                        sub = plsc.load_gather(sub_v, [ridx])
                        val = plsc.load_gather(rows_v, [ridx, sub * 2 + fbit])
                        wv = plsc.load_gather(w_v, [ridx])
                        contrib = val * wv
                        if c == 0:
                            plsc.store_scatter(enc_v, [pidx, cidx], contrib)
                        else:
                            plsc.addupdate_scatter(enc_v, [pidx, cidx], contrib)
                    return c2

                lax.fori_loop(0, CHUNK // 8, acc_body, 0)

            pltpu.sync_copy(enc_v, out_hbm.at[pl.ds(base + pbase, CHUNK)])
            return carry

        lax.fori_loop(0, NCHUNK, chunk_body, 0)

    return encode


def _head_body(enc_ref, pos_ref, w1s_ref, w2sd_ref, w2sr_ref, w1n_ref,
               w2n_ref, amn_ref, asc_ref, rgb_ref, den_ref, nrm_ref):
    dot = functools.partial(
        lax.dot_general,
        dimension_numbers=(((1,), (0,)), ((), ())),
        preferred_element_type=jnp.float32,
        precision=lax.Precision.HIGHEST,
    )
    enc = enc_ref[...]
    p = pos_ref[...]
    hs = jnp.maximum(dot(enc, w1s_ref[...]), 0.0)
    d0 = dot(hs, w2sd_ref[...])
    rgb_lin = dot(hs, w2sr_ref[...])
    hn = jnp.maximum(dot(enc, w1n_ref[...]), 0.0)
    nrm_ref[...] = dot(hn, w2n_ref[...])
    p2 = jnp.sum(p * p, axis=1, keepdims=True)
    bias = 10.0 * (1.0 - jnp.sqrt(p2) / 0.5)
    x = (p - amn_ref[...]) / asc_ref[...]
    inb = jnp.logical_and(x > 0.0, x < 1.0).astype(jnp.float32)
    sel = jnp.min(inb, axis=1, keepdims=True)
    den_ref[...] = jnp.exp(d0 + bias - 1.0) * sel
    rgb_ref[...] = 1.0 / (1.0 + jnp.exp(-rgb_lin))


def _tc_head(enc, positions, w1s, w2sd, w2sr, w1n, w2n, amn, asc):
    bn = 2048
    grid = N_PTS // bn
    full = lambda shape: pl.BlockSpec(shape, lambda i: (0, 0))
    return pl.pallas_call(
        _head_body,
        grid=(grid,),
        in_specs=[
            pl.BlockSpec((bn, 2 * N_LEVELS), lambda i: (i, 0)),
            pl.BlockSpec((bn, 3), lambda i: (i, 0)),
            full((32, 32)),
            full((32, 1)),
            full((32, 3)),
            full((32, 32)),
            full((32, 3)),
            full((1, 3)),
            full((1, 3)),
        ],
        out_specs=[
            pl.BlockSpec((bn, 3), lambda i: (i, 0)),
            pl.BlockSpec((bn, 1), lambda i: (i, 0)),
            pl.BlockSpec((bn, 3), lambda i: (i, 0)),
        ],
        out_shape=[
            jax.ShapeDtypeStruct((N_PTS, 3), jnp.float32),
            jax.ShapeDtypeStruct((N_PTS, 1), jnp.float32),
            jax.ShapeDtypeStruct((N_PTS, 3), jnp.float32),
        ],
    )(enc, positions, w1s, w2sd, w2sr, w1n, w2n, amn, asc)


def kernel(positions, directions, tables, W1s, W2s, W1n, W2n, aabb):
    pos_t = positions.T
    table_flat = tables.reshape(N_LEVELS * TBL * FEAT // 16, 16)
    enc = _sc_encode_fn()(pos_t, table_flat)
    amn = aabb[:3].reshape(1, 3)
    asc = (aabb[3:] - aabb[:3]).reshape(1, 3)
    rgb, den, nrm = _tc_head(enc, positions, W1s, W2s[:, :1], W2s[:, 1:],
                             W1n, W2n, amn, asc)
    return rgb, den, nrm


# speedup vs baseline: 1.0002x; 1.0002x over previous
import functools

import jax
import jax.numpy as jnp
import numpy as np
from jax import lax
from jax.experimental import pallas as pl
from jax.experimental.pallas import tpu as pltpu
from jax.experimental.pallas import tpu_sc as plsc

N_PTS = 131072
N_LEVELS = 16
FEAT = 2
TBL = 2 ** 19
BASE_RES = 16
PLS = 1.4472692012786865
MASK = TBL - 1
P1 = np.uint32(2654435761).astype(np.int32)
P2 = np.uint32(805459861).astype(np.int32)

NW = 32
PPW = N_PTS // NW
CHUNK = 128
NCHUNK = PPW // CHUNK

_RES = [int(np.floor(BASE_RES * (PLS ** l))) for l in range(N_LEVELS)]


@functools.cache
def _sc_encode_fn():
    mesh = plsc.VectorSubcoreMesh(core_axis_name="c", subcore_axis_name="s")

    @functools.partial(
        pl.kernel,
        mesh=mesh,
        compiler_params=pltpu.CompilerParams(
            needs_layout_passes=False, use_tc_tiling_on_sc=False),
        out_type=jax.ShapeDtypeStruct((N_PTS, 2 * N_LEVELS), jnp.float32),
        scratch_types=[
            pltpu.VMEM((3, PPW), jnp.float32),
            pltpu.VMEM((8, CHUNK), jnp.int32),
            pltpu.VMEM((8 * CHUNK,), jnp.int32),
            pltpu.VMEM((8 * CHUNK,), jnp.float32),
            pltpu.VMEM((8 * CHUNK, 16), jnp.float32),
            pltpu.VMEM((CHUNK, 2 * N_LEVELS), jnp.float32),
            pltpu.SemaphoreType.DMA,
        ],
    )
    def encode(pos_hbm, tab_hbm, out_hbm, pos_v, idx_v, sub_v, w_v, rows_v,
               enc_v, sem):
        cid = lax.axis_index("c")
        sid = lax.axis_index("s")
        wid = sid * 2 + cid
        base = wid * PPW

        pltpu.sync_copy(pos_hbm.at[:, pl.ds(base, PPW)], pos_v)

        lanes = lax.iota(jnp.int32, 16)
        half = lanes >> 1
        fbit = lanes & 1

        def norm_body(g, carry):
            for r in range(3):
                v = pos_v[r, pl.ds(g * 16, 16)]
                pos_v[r, pl.ds(g * 16, 16)] = (v + 1.0) * 0.5
            return carry

        lax.fori_loop(0, PPW // 16, norm_body, 0)

        def chunk_body(ch, carry):
            pbase = ch * CHUNK

            for l in range(N_LEVELS):
                res = float(_RES[l])
                lbase = l * TBL

                def hash_body(g, c2, res=res, lbase=lbase):
                    o = pbase + g * 16
                    ws = []
                    hs = []
                    for r, (mulc, addc) in enumerate(((1, 1), (int(P1), int(P1)), (int(P2), int(P2)))):
                        x = pos_v[r, pl.ds(o, 16)]
                        pos = x * res
                        ipos = pos.astype(jnp.int32)
                        w = pos - ipos.astype(jnp.float32)
                        sw = w * w * (3.0 - 2.0 * w)
                        ws.append((1.0 - sw, sw))
                        h0 = ipos * mulc if r > 0 else ipos
                        hs.append((h0, h0 + addc))
                    for c in range(8):
                        bx, by, bz = (c >> 0) & 1, (c >> 1) & 1, (c >> 2) & 1
                        h = ((hs[0][bx] ^ hs[1][by] ^ hs[2][bz]) & MASK) + lbase
                        idx_v[c, pl.ds(g * 16, 16)] = h >> 3
                        sub_v[pl.ds(c * CHUNK + g * 16, 16)] = h & 7
                        w_v[pl.ds(c * CHUNK + g * 16, 16)] = ws[0][bx] * ws[1][by] * ws[2][bz]
                    return c2

                lax.fori_loop(0, CHUNK // 16, hash_body, 0)

                handles = [
                    pltpu.async_copy(tab_hbm.at[idx_v.at[c]],
                                     rows_v.at[pl.ds(c * CHUNK, CHUNK)], sem)
                    for c in range(8)
                ]
                for h in handles:
                    h.wait()

                def acc_body(pg, c2, l=l):
                    pidx = pg * 8 + half
                    cidx = fbit + 2 * l
                    for c in range(8):
                        ridx = pidx + (c * CHUNK)
                        sub = plsc.load_gather(sub_v, [ridx])
                        val = plsc.load_gather(rows_v, [ridx, sub * 2 + fbit])
                        wv = plsc.load_gather(w_v, [ridx])
                        contrib = val * wv
                        if c == 0:
                            plsc.store_scatter(enc_v, [pidx, cidx], contrib)
                        else:
                            plsc.addupdate_scatter(enc_v, [pidx, cidx], contrib)
                    return c2

                lax.fori_loop(0, CHUNK // 8, acc_body, 0)

            pltpu.sync_copy(enc_v, out_hbm.at[pl.ds(base + pbase, CHUNK)])
            return carry

        lax.fori_loop(0, NCHUNK, chunk_body, 0)

    return encode


def _head_body(enc_ref, pos_ref, w1s_ref, w2sd_ref, w2sr_ref, w1n_ref,
               w2n_ref, amn_ref, asc_ref, rgb_ref, den_ref, nrm_ref):
    dot = functools.partial(
        lax.dot_general,
        dimension_numbers=(((1,), (0,)), ((), ())),
        preferred_element_type=jnp.float32,
        precision=lax.Precision.HIGHEST,
    )
    enc = enc_ref[...]
    p = pos_ref[...]
    hs = jnp.maximum(dot(enc, w1s_ref[...]), 0.0)
    d0 = dot(hs, w2sd_ref[...])
    rgb_lin = dot(hs, w2sr_ref[...])
    hn = jnp.maximum(dot(enc, w1n_ref[...]), 0.0)
    nrm_ref[...] = dot(hn, w2n_ref[...])
    p2 = jnp.sum(p * p, axis=1, keepdims=True)
    bias = 10.0 * (1.0 - jnp.sqrt(p2) / 0.5)
    x = (p - amn_ref[...]) / asc_ref[...]
    inb = jnp.logical_and(x > 0.0, x < 1.0).astype(jnp.float32)
    sel = jnp.min(inb, axis=1, keepdims=True)
    den_ref[...] = jnp.exp(d0 + bias - 1.0) * sel
    rgb_ref[...] = 1.0 / (1.0 + jnp.exp(-rgb_lin))


def _tc_head(enc, positions, w1s, w2sd, w2sr, w1n, w2n, amn, asc):
    bn = 2048
    grid = N_PTS // bn
    full = lambda shape: pl.BlockSpec(shape, lambda i: (0, 0))
    return pl.pallas_call(
        _head_body,
        grid=(grid,),
        in_specs=[
            pl.BlockSpec((bn, 2 * N_LEVELS), lambda i: (i, 0)),
            pl.BlockSpec((bn, 3), lambda i: (i, 0)),
            full((32, 32)),
            full((32, 1)),
            full((32, 3)),
            full((32, 32)),
            full((32, 3)),
            full((1, 3)),
            full((1, 3)),
        ],
        out_specs=[
            pl.BlockSpec((bn, 3), lambda i: (i, 0)),
            pl.BlockSpec((bn, 1), lambda i: (i, 0)),
            pl.BlockSpec((bn, 3), lambda i: (i, 0)),
        ],
        out_shape=[
            jax.ShapeDtypeStruct((N_PTS, 3), jnp.float32),
            jax.ShapeDtypeStruct((N_PTS, 1), jnp.float32),
            jax.ShapeDtypeStruct((N_PTS, 3), jnp.float32),
        ],
    )(enc, positions, w1s, w2sd, w2sr, w1n, w2n, amn, asc)


def kernel(positions, directions, tables, W1s, W2s, W1n, W2n, aabb):
    pos_t = positions.T
    table_lin = jax.lax.optimization_barrier(
        tables.reshape(N_LEVELS * TBL * FEAT // 128, 128))
    table_flat = table_lin.reshape(N_LEVELS * TBL * FEAT // 16, 16)
    enc = _sc_encode_fn()(pos_t, table_flat)
    amn = aabb[:3].reshape(1, 3)
    asc = (aabb[3:] - aabb[:3]).reshape(1, 3)
    rgb, den, nrm = _tc_head(enc, positions, W1s, W2s[:, :1], W2s[:, 1:],
                             W1n, W2n, amn, asc)
    return rgb, den, nrm


# speedup vs baseline: 4.8506x; 4.8494x over previous
import functools

import jax
import jax.numpy as jnp
import numpy as np
from jax import lax
from jax.experimental import pallas as pl
from jax.experimental.pallas import tpu as pltpu
from jax.experimental.pallas import tpu_sc as plsc

N_PTS = 131072
N_LEVELS = 16
FEAT = 2
TBL = 2 ** 19
BASE_RES = 16
PLS = 1.4472692012786865
MASK = TBL - 1
P1 = np.uint32(2654435761).astype(np.int32)
P2 = np.uint32(805459861).astype(np.int32)

NW = 32
PPW = N_PTS // NW
CHUNK = 128
NCHUNK = PPW // CHUNK

_RES = [int(np.floor(BASE_RES * (PLS ** l))) for l in range(N_LEVELS)]


@functools.cache
def _sc_convert_fn():
    mesh = plsc.VectorSubcoreMesh(core_axis_name="c", subcore_axis_name="s")
    nblk = 64
    total = N_LEVELS * TBL * FEAT

    @functools.partial(
        pl.kernel,
        mesh=mesh,
        compiler_params=pltpu.CompilerParams(
            needs_layout_passes=False, use_tc_tiling_on_sc=False),
        out_type=jax.ShapeDtypeStruct((total,), jnp.float32),
        scratch_types=[
            pltpu.VMEM((nblk * 256,), jnp.float32),
            pltpu.VMEM((nblk * 256,), jnp.float32),
        ],
    )
    def convert(src_hbm, out_hbm, buf_i, buf_o):
        cid = lax.axis_index("c")
        sid = lax.axis_index("s")
        wid = sid * 2 + cid
        epw = total // NW
        ebase = wid * epw
        lanes2 = lax.iota(jnp.int32, 16) * 2

        def it_body(t, carry):
            off = ebase + t * (nblk * 256)
            pltpu.sync_copy(src_hbm.at[pl.ds(off, nblk * 256)], buf_i)

            def blk_body(b, c2):
                ib = b * 256
                for j in range(8):
                    v0 = buf_i[pl.ds(ib + j * 16, 16)]
                    v1 = buf_i[pl.ds(ib + 128 + j * 16, 16)]
                    oidx = ib + j * 32 + lanes2
                    plsc.store_scatter(buf_o, [oidx], v0)
                    plsc.store_scatter(buf_o, [oidx + 1], v1)
                return c2

            lax.fori_loop(0, nblk, blk_body, 0)
            pltpu.sync_copy(buf_o, out_hbm.at[pl.ds(off, nblk * 256)])
            return carry

        lax.fori_loop(0, epw // (nblk * 256), it_body, 0)

    return convert


@functools.cache
def _sc_encode_fn():
    mesh = plsc.VectorSubcoreMesh(core_axis_name="c", subcore_axis_name="s")

    @functools.partial(
        pl.kernel,
        mesh=mesh,
        compiler_params=pltpu.CompilerParams(
            needs_layout_passes=False, use_tc_tiling_on_sc=False),
        out_type=jax.ShapeDtypeStruct((N_PTS, 2 * N_LEVELS), jnp.float32),
        scratch_types=[
            pltpu.VMEM((3, PPW), jnp.float32),
            pltpu.VMEM((8, CHUNK), jnp.int32),
            pltpu.VMEM((8 * CHUNK,), jnp.int32),
            pltpu.VMEM((8 * CHUNK,), jnp.float32),
            pltpu.VMEM((8 * CHUNK, 16), jnp.float32),
            pltpu.VMEM((CHUNK, 2 * N_LEVELS), jnp.float32),
            pltpu.SemaphoreType.DMA,
        ],
    )
    def encode(pos_hbm, tab_hbm, out_hbm, pos_v, idx_v, sub_v, w_v, rows_v,
               enc_v, sem):
        cid = lax.axis_index("c")
        sid = lax.axis_index("s")
        wid = sid * 2 + cid
        base = wid * PPW

        pltpu.sync_copy(pos_hbm.at[:, pl.ds(base, PPW)], pos_v)

        lanes = lax.iota(jnp.int32, 16)
        half = lanes >> 1
        fbit = lanes & 1

        def norm_body(g, carry):
            for r in range(3):
                v = pos_v[r, pl.ds(g * 16, 16)]
                pos_v[r, pl.ds(g * 16, 16)] = (v + 1.0) * 0.5
            return carry

        lax.fori_loop(0, PPW // 16, norm_body, 0)

        def chunk_body(ch, carry):
            pbase = ch * CHUNK

            for l in range(N_LEVELS):
                res = float(_RES[l])
                lbase = l * TBL

                def hash_body(g, c2, res=res, lbase=lbase):
                    o = pbase + g * 16
                    ws = []
                    hs = []
                    for r, (mulc, addc) in enumerate(((1, 1), (int(P1), int(P1)), (int(P2), int(P2)))):
                        x = pos_v[r, pl.ds(o, 16)]
                        pos = x * res
                        ipos = pos.astype(jnp.int32)
                        w = pos - ipos.astype(jnp.float32)
                        sw = w * w * (3.0 - 2.0 * w)
                        ws.append((1.0 - sw, sw))
                        h0 = ipos * mulc if r > 0 else ipos
                        hs.append((h0, h0 + addc))
                    for c in range(8):
                        bx, by, bz = (c >> 0) & 1, (c >> 1) & 1, (c >> 2) & 1
                        h = ((hs[0][bx] ^ hs[1][by] ^ hs[2][bz]) & MASK) + lbase
                        idx_v[c, pl.ds(g * 16, 16)] = h >> 3
                        sub_v[pl.ds(c * CHUNK + g * 16, 16)] = h & 7
                        w_v[pl.ds(c * CHUNK + g * 16, 16)] = ws[0][bx] * ws[1][by] * ws[2][bz]
                    return c2

                lax.fori_loop(0, CHUNK // 16, hash_body, 0)

                handles = [
                    pltpu.async_copy(tab_hbm.at[idx_v.at[c]],
                                     rows_v.at[pl.ds(c * CHUNK, CHUNK)], sem)
                    for c in range(8)
                ]
                for h in handles:
                    h.wait()

                def acc_body(pg, c2, l=l):
                    pidx = pg * 8 + half
                    cidx = fbit + 2 * l
                    for c in range(8):
                        ridx = pidx + (c * CHUNK)
                        sub = plsc.load_gather(sub_v, [ridx])
                        val = plsc.load_gather(rows_v, [ridx, sub * 2 + fbit])
                        wv = plsc.load_gather(w_v, [ridx])
                        contrib = val * wv
                        if c == 0:
                            plsc.store_scatter(enc_v, [pidx, cidx], contrib)
                        else:
                            plsc.addupdate_scatter(enc_v, [pidx, cidx], contrib)
                    return c2

                lax.fori_loop(0, CHUNK // 8, acc_body, 0)

            pltpu.sync_copy(enc_v, out_hbm.at[pl.ds(base + pbase, CHUNK)])
            return carry

        lax.fori_loop(0, NCHUNK, chunk_body, 0)

    return encode


def _head_body(enc_ref, pos_ref, w1s_ref, w2sd_ref, w2sr_ref, w1n_ref,
               w2n_ref, amn_ref, asc_ref, rgb_ref, den_ref, nrm_ref):
    dot = functools.partial(
        lax.dot_general,
        dimension_numbers=(((1,), (0,)), ((), ())),
        preferred_element_type=jnp.float32,
        precision=lax.Precision.HIGHEST,
    )
    enc = enc_ref[...]
    p = pos_ref[...]
    hs = jnp.maximum(dot(enc, w1s_ref[...]), 0.0)
    d0 = dot(hs, w2sd_ref[...])
    rgb_lin = dot(hs, w2sr_ref[...])
    hn = jnp.maximum(dot(enc, w1n_ref[...]), 0.0)
    nrm_ref[...] = dot(hn, w2n_ref[...])
    p2 = jnp.sum(p * p, axis=1, keepdims=True)
    bias = 10.0 * (1.0 - jnp.sqrt(p2) / 0.5)
    x = (p - amn_ref[...]) / asc_ref[...]
    inb = jnp.logical_and(x > 0.0, x < 1.0).astype(jnp.float32)
    sel = jnp.min(inb, axis=1, keepdims=True)
    den_ref[...] = jnp.exp(d0 + bias - 1.0) * sel
    rgb_ref[...] = 1.0 / (1.0 + jnp.exp(-rgb_lin))


def _tc_head(enc, positions, w1s, w2sd, w2sr, w1n, w2n, amn, asc):
    bn = 2048
    grid = N_PTS // bn
    full = lambda shape: pl.BlockSpec(shape, lambda i: (0, 0))
    return pl.pallas_call(
        _head_body,
        grid=(grid,),
        in_specs=[
            pl.BlockSpec((bn, 2 * N_LEVELS), lambda i: (i, 0)),
            pl.BlockSpec((bn, 3), lambda i: (i, 0)),
            full((32, 32)),
            full((32, 1)),
            full((32, 3)),
            full((32, 32)),
            full((32, 3)),
            full((1, 3)),
            full((1, 3)),
        ],
        out_specs=[
            pl.BlockSpec((bn, 3), lambda i: (i, 0)),
            pl.BlockSpec((bn, 1), lambda i: (i, 0)),
            pl.BlockSpec((bn, 3), lambda i: (i, 0)),
        ],
        out_shape=[
            jax.ShapeDtypeStruct((N_PTS, 3), jnp.float32),
            jax.ShapeDtypeStruct((N_PTS, 1), jnp.float32),
            jax.ShapeDtypeStruct((N_PTS, 3), jnp.float32),
        ],
    )(enc, positions, w1s, w2sd, w2sr, w1n, w2n, amn, asc)


def kernel(positions, directions, tables, W1s, W2s, W1n, W2n, aabb):
    pos_t = positions.T
    table_native = tables.reshape(
        N_LEVELS, TBL // 128, 128, FEAT).transpose(0, 1, 3, 2).reshape(-1)
    table_lin = _sc_convert_fn()(table_native)
    table_flat = table_lin.reshape(N_LEVELS * TBL * FEAT // 16, 16)
    enc = _sc_encode_fn()(pos_t, table_flat)
    amn = aabb[:3].reshape(1, 3)
    asc = (aabb[3:] - aabb[:3]).reshape(1, 3)
    rgb, den, nrm = _tc_head(enc, positions, W1s, W2s[:, :1], W2s[:, 1:],
                             W1n, W2n, amn, asc)
    return rgb, den, nrm


# speedup vs baseline: 7.1595x; 1.4760x over previous
import functools

import jax
import jax.numpy as jnp
import numpy as np
from jax import lax
from jax.experimental import pallas as pl
from jax.experimental.pallas import tpu as pltpu
from jax.experimental.pallas import tpu_sc as plsc

N_PTS = 131072
N_LEVELS = 16
FEAT = 2
TBL = 2 ** 19
BASE_RES = 16
PLS = 1.4472692012786865
MASK = TBL - 1
P1 = np.uint32(2654435761).astype(np.int32)
P2 = np.uint32(805459861).astype(np.int32)

NW = 32
PPW = N_PTS // NW
CHUNK = 128
NCHUNK = PPW // CHUNK

_RES = [int(np.floor(BASE_RES * (PLS ** l))) for l in range(N_LEVELS)]


@functools.cache
def _sc_convert_fn():
    mesh = plsc.VectorSubcoreMesh(core_axis_name="c", subcore_axis_name="s")
    nblk = 64
    total = N_LEVELS * TBL * FEAT

    @functools.partial(
        pl.kernel,
        mesh=mesh,
        compiler_params=pltpu.CompilerParams(
            needs_layout_passes=False, use_tc_tiling_on_sc=False),
        out_type=jax.ShapeDtypeStruct((total,), jnp.float32),
        scratch_types=[
            pltpu.VMEM((nblk * 256,), jnp.float32),
            pltpu.VMEM((nblk * 256,), jnp.float32),
        ],
    )
    def convert(src_hbm, out_hbm, buf_i, buf_o):
        cid = lax.axis_index("c")
        sid = lax.axis_index("s")
        wid = sid * 2 + cid
        epw = total // NW
        ebase = wid * epw
        lanes2 = lax.iota(jnp.int32, 16) * 2

        def it_body(t, carry):
            off = ebase + t * (nblk * 256)
            pltpu.sync_copy(src_hbm.at[pl.ds(off, nblk * 256)], buf_i)

            def blk_body(b, c2):
                ib = b * 256
                for j in range(8):
                    v0 = buf_i[pl.ds(ib + j * 16, 16)]
                    v1 = buf_i[pl.ds(ib + 128 + j * 16, 16)]
                    oidx = ib + j * 32 + lanes2
                    plsc.store_scatter(buf_o, [oidx], v0)
                    plsc.store_scatter(buf_o, [oidx + 1], v1)
                return c2

            lax.fori_loop(0, nblk, blk_body, 0)
            pltpu.sync_copy(buf_o, out_hbm.at[pl.ds(off, nblk * 256)])
            return carry

        lax.fori_loop(0, epw // (nblk * 256), it_body, 0)

    return convert


@functools.cache
def _sc_encode_fn():
    mesh = plsc.VectorSubcoreMesh(core_axis_name="c", subcore_axis_name="s")

    @functools.partial(
        pl.kernel,
        mesh=mesh,
        compiler_params=pltpu.CompilerParams(
            needs_layout_passes=False, use_tc_tiling_on_sc=False),
        out_type=jax.ShapeDtypeStruct((N_PTS, 2 * N_LEVELS), jnp.float32),
        scratch_types=[
            pltpu.VMEM((3, PPW), jnp.float32),
            pltpu.VMEM((8, CHUNK), jnp.int32),
            pltpu.VMEM((8, CHUNK), jnp.int32),
            pltpu.VMEM((8 * CHUNK,), jnp.float32),
            pltpu.VMEM((8 * CHUNK,), jnp.float32),
            pltpu.VMEM((8 * CHUNK, 16), jnp.float32),
            pltpu.VMEM((8 * CHUNK, 16), jnp.float32),
            pltpu.VMEM((CHUNK, 2 * N_LEVELS), jnp.float32),
            pltpu.SemaphoreType.DMA,
            pltpu.SemaphoreType.DMA,
        ],
    )
    def encode(pos_hbm, tab_hbm, out_hbm, pos_v, idx_a, idx_b, w_a, w_b,
               rows_a, rows_b, enc_v, sem_a, sem_b):
        cid = lax.axis_index("c")
        sid = lax.axis_index("s")
        wid = sid * 2 + cid
        base = wid * PPW

        pltpu.sync_copy(pos_hbm.at[:, pl.ds(base, PPW)], pos_v)

        lanes = lax.iota(jnp.int32, 16)
        half = lanes >> 1
        fbit = lanes & 1
        idx_bufs = (idx_a, idx_b)
        w_bufs = (w_a, w_b)
        rows_bufs = (rows_a, rows_b)
        sems = (sem_a, sem_b)

        def norm_body(g, carry):
            for r in range(3):
                v = pos_v[r, pl.ds(g * 16, 16)]
                pos_v[r, pl.ds(g * 16, 16)] = (v + 1.0) * 0.5
            return carry

        lax.fori_loop(0, PPW // 16, norm_body, 0)

        def chunk_body(ch, carry):
            pbase = ch * CHUNK

            def hash_level(l):
                res = float(_RES[l])
                lbase = l * TBL
                idx_v = idx_bufs[l & 1]
                w_v = w_bufs[l & 1]

                def hash_body(g, c2):
                    o = pbase + g * 16
                    ws = []
                    hs = []
                    for r, (mulc, addc) in enumerate(
                            ((1, 1), (int(P1), int(P1)), (int(P2), int(P2)))):
                        x = pos_v[r, pl.ds(o, 16)]
                        pos = x * res
                        ipos = pos.astype(jnp.int32)
                        w = pos - ipos.astype(jnp.float32)
                        sw = w * w * (3.0 - 2.0 * w)
                        ws.append((1.0 - sw, sw))
                        h0 = ipos * mulc if r > 0 else ipos
                        hs.append((h0, h0 + addc))
                    for c in range(8):
                        bx, by, bz = (c >> 0) & 1, (c >> 1) & 1, (c >> 2) & 1
                        h = ((hs[0][bx] ^ hs[1][by] ^ hs[2][bz]) & MASK) + lbase
                        idx_v[c, pl.ds(g * 16, 16)] = h >> 3
                        w3 = ws[0][bx] * ws[1][by] * ws[2][bz]
                        wi = plsc.bitcast(w3, jnp.int32)
                        w_v[pl.ds(c * CHUNK + g * 16, 16)] = plsc.bitcast(
                            (wi & ~7) | (h & 7), jnp.float32)
                    return c2

                lax.fori_loop(0, CHUNK // 16, hash_body, 0)

            def fire_level(l):
                idx_v = idx_bufs[l & 1]
                rows_v = rows_bufs[l & 1]
                return [
                    pltpu.async_copy(tab_hbm.at[idx_v.at[c]],
                                     rows_v.at[pl.ds(c * CHUNK, CHUNK)],
                                     sems[l & 1])
                    for c in range(8)
                ]

            def acc_level(l):
                w_v = w_bufs[l & 1]
                rows_v = rows_bufs[l & 1]

                def acc_body(pg, c2):
                    pidx = pg * 8 + half
                    cidx = fbit + 2 * l
                    for c in range(8):
                        ridx = pidx + (c * CHUNK)
                        wv = plsc.load_gather(w_v, [ridx])
                        sub = plsc.bitcast(wv, jnp.int32) & 7
                        val = plsc.load_gather(rows_v, [ridx, sub * 2 + fbit])
                        contrib = val * wv
                        if c == 0:
                            plsc.store_scatter(enc_v, [pidx, cidx], contrib)
                        else:
                            plsc.addupdate_scatter(enc_v, [pidx, cidx], contrib)
                    return c2

                lax.fori_loop(0, CHUNK // 8, acc_body, 0)

            hash_level(0)
            handles = fire_level(0)
            for l in range(N_LEVELS):
                if l + 1 < N_LEVELS:
                    hash_level(l + 1)
                    nxt = fire_level(l + 1)
                for h in handles:
                    h.wait()
                acc_level(l)
                if l + 1 < N_LEVELS:
                    handles = nxt

            pltpu.sync_copy(enc_v, out_hbm.at[pl.ds(base + pbase, CHUNK)])
            return carry

        lax.fori_loop(0, NCHUNK, chunk_body, 0)

    return encode


def _head_body(enc_ref, pos_ref, w1s_ref, w2sd_ref, w2sr_ref, w1n_ref,
               w2n_ref, amn_ref, asc_ref, rgb_ref, den_ref, nrm_ref):
    dot = functools.partial(
        lax.dot_general,
        dimension_numbers=(((1,), (0,)), ((), ())),
        preferred_element_type=jnp.float32,
        precision=lax.Precision.HIGHEST,
    )
    enc = enc_ref[...]
    p = pos_ref[...]
    hs = jnp.maximum(dot(enc, w1s_ref[...]), 0.0)
    d0 = dot(hs, w2sd_ref[...])
    rgb_lin = dot(hs, w2sr_ref[...])
    hn = jnp.maximum(dot(enc, w1n_ref[...]), 0.0)
    nrm_ref[...] = dot(hn, w2n_ref[...])
    p2 = jnp.sum(p * p, axis=1, keepdims=True)
    bias = 10.0 * (1.0 - jnp.sqrt(p2) / 0.5)
    x = (p - amn_ref[...]) / asc_ref[...]
    inb = jnp.logical_and(x > 0.0, x < 1.0).astype(jnp.float32)
    sel = jnp.min(inb, axis=1, keepdims=True)
    den_ref[...] = jnp.exp(d0 + bias - 1.0) * sel
    rgb_ref[...] = 1.0 / (1.0 + jnp.exp(-rgb_lin))


def _tc_head(enc, positions, w1s, w2sd, w2sr, w1n, w2n, amn, asc):
    bn = 2048
    grid = N_PTS // bn
    full = lambda shape: pl.BlockSpec(shape, lambda i: (0, 0))
    return pl.pallas_call(
        _head_body,
        grid=(grid,),
        in_specs=[
            pl.BlockSpec((bn, 2 * N_LEVELS), lambda i: (i, 0)),
            pl.BlockSpec((bn, 3), lambda i: (i, 0)),
            full((32, 32)),
            full((32, 1)),
            full((32, 3)),
            full((32, 32)),
            full((32, 3)),
            full((1, 3)),
            full((1, 3)),
        ],
        out_specs=[
            pl.BlockSpec((bn, 3), lambda i: (i, 0)),
            pl.BlockSpec((bn, 1), lambda i: (i, 0)),
            pl.BlockSpec((bn, 3), lambda i: (i, 0)),
        ],
        out_shape=[
            jax.ShapeDtypeStruct((N_PTS, 3), jnp.float32),
            jax.ShapeDtypeStruct((N_PTS, 1), jnp.float32),
            jax.ShapeDtypeStruct((N_PTS, 3), jnp.float32),
        ],
    )(enc, positions, w1s, w2sd, w2sr, w1n, w2n, amn, asc)


def kernel(positions, directions, tables, W1s, W2s, W1n, W2n, aabb):
    pos_t = positions.T
    table_native = tables.reshape(
        N_LEVELS, TBL // 128, 128, FEAT).transpose(0, 1, 3, 2).reshape(-1)
    table_lin = _sc_convert_fn()(table_native)
    table_flat = table_lin.reshape(N_LEVELS * TBL * FEAT // 16, 16)
    enc = _sc_encode_fn()(pos_t, table_flat)
    amn = aabb[:3].reshape(1, 3)
    asc = (aabb[3:] - aabb[:3]).reshape(1, 3)
    rgb, den, nrm = _tc_head(enc, positions, W1s, W2s[:, :1], W2s[:, 1:],
                             W1n, W2n, amn, asc)
    return rgb, den, nrm
